# baseline (speedup 1.0000x reference)
import functools

import jax
import jax.numpy as jnp
from jax import lax
from jax.experimental import pallas as pl
from jax.experimental.pallas import tpu as pltpu

B, H, S, D = 4, 8, 1024, 64
L = 1024
C_GAMMA = 0.99
EPS = 0.01
TS = 1024
T = S // TS
NH = 4
NHALF = H // NH

CHUNK = 128
NROW_H = B * NH * S
RPW = NROW_H // 32
NCH = RPW // CHUNK
HL = NH * L


def _a_body(vecs_ref, mask_ref, w_ref, cnt_ref,
            z_ref, errs_ref, cb_ref, gidx_ref, counts_ref, cbn_ref, eye_ref):
    h = pl.program_id(0)
    b = pl.program_id(1)

    @pl.when((h == 0) & (b == 0))
    def _eye():
        ri = lax.broadcasted_iota(jnp.int32, (TS, TS), 0)
        ci = lax.broadcasted_iota(jnp.int32, (TS, TS), 1)
        eye_ref[...] = jnp.where(ri == ci, 1.0, 0.0).astype(jnp.float32)

    @pl.when(b == 0)
    def _init():
        cb = w_ref[0] / (cnt_ref[0, 0][:, None] + EPS)
        cb_ref[0] = cb
        cbn_ref[...] = jnp.sum(cb * cb, axis=1, keepdims=True)
        counts_ref[0] = jnp.zeros((L, 1), jnp.float32)

    cb = cb_ref[0]
    cbn = cbn_ref[...]
    v = vecs_ref[0, 0]
    vn = jnp.sum(v * v, axis=1, keepdims=True)
    vn_lane = lax.dot_general(vn, eye_ref[...], (((0,), (0,)), ((), ())),
                              preferred_element_type=jnp.float32)
    scores_t = lax.dot_general(cb, v, (((1,), (1,)), ((), ())),
                               preferred_element_type=jnp.float32)
    d2 = (vn_lane - 2.0 * scores_t) + cbn
    m = jnp.min(d2, axis=0)
    iota = lax.broadcasted_iota(jnp.int32, (L, TS), 0)
    z = jnp.argmin(d2, axis=0).astype(jnp.int32)
    z_ref[0, 0, 0, 0] = z
    gidx_ref[0, 0, 0, 0] = z + h * L
    errs_ref[0, 0, 0, 0] = m
    msk = mask_ref[0, 0, 0]
    onehot_t = jnp.where(z[None, :] == iota, msk[None, :], 0.0)
    ones_row = jnp.ones((1, TS), jnp.float32)
    counts_ref[0] += lax.dot_general(
        onehot_t, ones_row, (((1,), (1,)), ((), ())),
        preferred_element_type=jnp.float32)


def _run_a(vecs, mask4, w, c_count3, h_base):
    grid = (NH, B * T)
    out_shapes = (
        jax.ShapeDtypeStruct((B, NH, T, 1, TS), jnp.int32),
        jax.ShapeDtypeStruct((B, NH, T, 1, TS), jnp.float32),
        jax.ShapeDtypeStruct((NH, L, D), jnp.float32),
        jax.ShapeDtypeStruct((B, NH, T, 1, TS), jnp.int32),
        jax.ShapeDtypeStruct((NH, L, 1), jnp.float32),
    )
    in_specs = [
        pl.BlockSpec((1, 1, TS, D), lambda h, b: (b // T, h_base + h, b % T, 0)),
        pl.BlockSpec((1, 1, 1, TS), lambda h, b: (b // T, b % T, 0, 0)),
        pl.BlockSpec((1, L, D), lambda h, b: (h_base + h, 0, 0)),
        pl.BlockSpec((1, 1, L), lambda h, b: (h_base + h, 0, 0)),
    ]
    out_specs = (
        pl.BlockSpec((1, 1, 1, 1, TS), lambda h, b: (b // T, h, b % T, 0, 0)),
        pl.BlockSpec((1, 1, 1, 1, TS), lambda h, b: (b // T, h, b % T, 0, 0)),
        pl.BlockSpec((1, L, D), lambda h, b: (h, 0, 0)),
        pl.BlockSpec((1, 1, 1, 1, TS), lambda h, b: (b // T, h, b % T, 0, 0)),
        pl.BlockSpec((1, L, 1), lambda h, b: (h, 0, 0)),
    )
    return pl.pallas_call(
        _a_body,
        grid=grid,
        in_specs=in_specs,
        out_specs=out_specs,
        out_shape=out_shapes,
        scratch_shapes=[pltpu.VMEM((L, 1), jnp.float32),
                        pltpu.VMEM((TS, TS), jnp.float32)],
    )(vecs, mask4, w, c_count3)


def _make_sc_kernel(kappa):
    from jax.experimental.pallas import tpu_sc as plsc

    mesh = plsc.VectorSubcoreMesh(core_axis_name="c", subcore_axis_name="s")

    @functools.partial(
        pl.kernel,
        mesh=mesh,
        out_type=[
            jax.ShapeDtypeStruct((NROW_H, D), jnp.float32),
            jax.ShapeDtypeStruct((2, HL, D), jnp.float32),
        ],
        scratch_types=[
            pltpu.VMEM((NCH, CHUNK), jnp.int32),
            pltpu.VMEM((NCH, CHUNK, D), jnp.float32),
            pltpu.VMEM((NCH, CHUNK, D), jnp.float32),
            pltpu.VMEM((CHUNK, D), jnp.float32),
            pltpu.VMEM_SHARED((HL, D), jnp.float32),
            pltpu.SemaphoreType.DMA,
            pltpu.SemaphoreType.DMA,
            pltpu.SemaphoreType.DMA,
            pltpu.SemaphoreType.DMA,
        ],
        compiler_params=pltpu.CompilerParams(use_tc_tiling_on_sc=False),
    )
    def sc_b(gidx_hbm, cb_hbm, vecs_hbm, vh_hbm, parts_hbm,
             idx_v, v_buf, cz_buf, zbuf, acc, sem_v, sem_g, sem_s, sem_a):
        c = lax.axis_index("c")
        s = lax.axis_index("s")
        wid = s * 2 + c
        pair = wid // 2
        shalf = wid % 2
        b_ = pair // NH
        hp = pair % NH
        gbase = (b_ * H + (kappa * NH + hp)) * S + shalf * RPW
        lbase = (b_ * NH + hp) * S + shalf * RPW

        def _zero_row(i, _):
            r = i // (D // 16)
            col = (i % (D // 16)) * 16
            zbuf[r, pl.ds(col, 16)] = jnp.zeros((16,), jnp.float32)
            return _
        lax.fori_loop(0, CHUNK * (D // 16), _zero_row, 0)
        pltpu.sync_copy(gidx_hbm.at[pl.ds(lbase // CHUNK, NCH)], idx_v)
        for k in range(HL // (16 * CHUNK)):
            pltpu.sync_copy(zbuf, acc.at[pl.ds(s * (HL // 16) + k * CHUNK, CHUNK)])
        plsc.subcore_barrier()

        vc, gc = [], []
        for j in range(NCH):
            vc.append(pltpu.async_copy(vecs_hbm.at[pl.ds(gbase + j * CHUNK, CHUNK)],
                                       v_buf.at[j], sem_v))
            gc.append(pltpu.async_copy(cb_hbm.at[idx_v.at[j]],
                                       cz_buf.at[j], sem_g))
        wr = []
        for j in range(NCH):
            gc[j].wait()
            wr.append(pltpu.async_copy(cz_buf.at[j],
                                       vh_hbm.at[pl.ds(lbase + j * CHUNK, CHUNK)],
                                       sem_s))
            vc[j].wait()
            wr.append(pltpu.async_copy(v_buf.at[j], acc.at[idx_v.at[j]],
                                       sem_a, add=True))
        for cpy in wr:
            cpy.wait()
        plsc.subcore_barrier()

        pltpu.sync_copy(acc.at[pl.ds(s * (HL // 16), HL // 16)],
                        parts_hbm.at[c, pl.ds(s * (HL // 16), HL // 16)])

    return sc_b


def _c_body(e1_ref, e2_ref, maskb4_ref, w1_ref, w2_ref, cc1_ref, cc2_ref,
            p1_ref, p2_ref, ch1_ref, ch2_ref, lcommit_ref, lcb_ref):
    msk = maskb4_ref[...]
    lc = (jnp.sum(msk * e1_ref[...]) + jnp.sum(msk * e2_ref[...])) \
        * (1.0 / (B * S))
    lcommit_ref[...] = lc.reshape(1, 1)
    total = jnp.float32(0.0)
    for w_ref, p_ref, cc_ref, ch_ref in (
            (w1_ref, p1_ref, cc1_ref, ch1_ref),
            (w2_ref, p2_ref, cc2_ref, ch2_ref)):
        csum = p_ref[0] + p_ref[1]
        w2d = w_ref[...]
        tgt = (1.0 - C_GAMMA) * w2d + C_GAMMA * csum
        total += jnp.sum(jnp.square(w2d - tgt))
        cc = cc_ref[...]
        ctgt = (1.0 - C_GAMMA) * cc + C_GAMMA * ch_ref[...]
        total += jnp.sum(jnp.square(cc - ctgt))
    lcb_ref[...] = total.reshape(1, 1)


def _run_c(e1, e2, maskb4, w1, w2, cc1, cc2, p1, p2, ch1, ch2):
    return pl.pallas_call(
        _c_body,
        out_shape=(jax.ShapeDtypeStruct((1, 1), jnp.float32),
                   jax.ShapeDtypeStruct((1, 1), jnp.float32)),
    )(e1, e2, maskb4, w1, w2, cc1, cc2, p1, p2, ch1, ch2)


def kernel(vecs, loss_mask, w, c_count):
    mask4 = loss_mask.reshape(B, T, 1, TS)
    c_count3 = c_count.reshape(H, 1, L)

    halves = []
    for kappa in range(NHALF):
        z5, errs5, cb, gidx5, counts3 = _run_a(vecs, mask4, w, c_count3,
                                               kappa * NH)
        gidx2d = gidx5.reshape(NROW_H // CHUNK, CHUNK)
        vh_flat, parts = _make_sc_kernel(kappa)(
            gidx2d, cb.reshape(HL, D), vecs.reshape(B * H * S, D))
        halves.append((z5, errs5, parts, counts3, vh_flat))

    (z5a, e5a, pa, ca, vha), (z5b, e5b, pb, cb_, vhb) = halves
    vecs_hat = jnp.concatenate(
        [vha.reshape(B, NH, S, D), vhb.reshape(B, NH, S, D)], axis=1)
    z = jnp.concatenate(
        [z5a.reshape(B, NH, S), z5b.reshape(B, NH, S)], axis=1)

    maskb4 = jnp.broadcast_to(loss_mask[:, None, :], (B, NH, S)).reshape(B * NH, S)
    lc, lcb = _run_c(
        e5a.reshape(B * NH, S), e5b.reshape(B * NH, S), maskb4,
        w[:NH].reshape(HL, D), w[NH:].reshape(HL, D),
        c_count[:NH], c_count[NH:],
        pa, pb, ca.reshape(NH, L), cb_.reshape(NH, L))
    return (vecs_hat, z, lc.reshape(()), lcb.reshape(()))


# speedup vs baseline: 1.0873x; 1.0873x over previous
import functools

import jax
import jax.numpy as jnp
from jax import lax
from jax.experimental import pallas as pl
from jax.experimental.pallas import tpu as pltpu

B, H, S, D = 4, 8, 1024, 64
L = 1024
C_GAMMA = 0.99
EPS = 0.01
TS = 1024
T = S // TS


def _a_body(vecs_ref, mask_ref, w_ref, cnt_ref,
            z_ref, errs_ref, cb_ref, gidx_ref, counts_ref,
            cbn_ref, eye_ref, vn_ref):
    b = pl.program_id(0)
    h = pl.program_id(1)

    @pl.when((b == 0) & (h == 0))
    def _init():
        ri = lax.broadcasted_iota(jnp.int32, (TS, TS), 0)
        ci = lax.broadcasted_iota(jnp.int32, (TS, TS), 1)
        eye_ref[...] = jnp.where(ri == ci, 1.0, 0.0).astype(jnp.float32)
        cb_all = w_ref[...] / (cnt_ref[...] + EPS)
        cb_ref[...] = cb_all
        cbn_ref[...] = jnp.sum(cb_all * cb_all, axis=2, keepdims=True)
        counts_ref[...] = jnp.zeros((H, L, 1), jnp.float32)

    v = vecs_ref[0, 0]

    @pl.when(h == 0)
    def _vn():
        vn = jnp.sum(v * v, axis=1, keepdims=True)
        vn_ref[...] = lax.dot_general(vn, eye_ref[...],
                                      (((0,), (0,)), ((), ())),
                                      preferred_element_type=jnp.float32)

    cb = cb_ref[h]
    cbn = cbn_ref[h]
    scores_t = lax.dot_general(cb, v, (((1,), (1,)), ((), ())),
                               preferred_element_type=jnp.float32)
    d2 = (vn_ref[...] - 2.0 * scores_t) + cbn
    m = jnp.min(d2, axis=0)
    iota = lax.broadcasted_iota(jnp.int32, (L, TS), 0)
    z = jnp.argmin(d2, axis=0).astype(jnp.int32)
    z_ref[0, 0, 0, 0] = z
    gidx_ref[0, 0, 0, 0] = z + h * L
    errs_ref[0, 0, 0, 0] = m
    msk = mask_ref[0, 0, 0]
    onehot_t = jnp.where(z[None, :] == iota, msk[None, :], 0.0)
    ones_row = jnp.ones((1, TS), jnp.float32)
    counts_ref[h] += lax.dot_general(
        onehot_t, ones_row, (((1,), (1,)), ((), ())),
        preferred_element_type=jnp.float32)


def _run_a(vecs, mask4, w, c_count):
    grid = (B * T, H)
    out_shapes = (
        jax.ShapeDtypeStruct((B, H, T, 1, TS), jnp.int32),
        jax.ShapeDtypeStruct((B, H, T, 1, TS), jnp.float32),
        jax.ShapeDtypeStruct((H, L, D), jnp.float32),
        jax.ShapeDtypeStruct((B, H, T, 1, TS), jnp.int32),
        jax.ShapeDtypeStruct((H, L, 1), jnp.float32),
    )
    in_specs = [
        pl.BlockSpec((1, 1, TS, D), lambda b, h: (b // T, h, b % T, 0)),
        pl.BlockSpec((1, 1, 1, TS), lambda b, h: (b // T, b % T, 0, 0)),
        pl.BlockSpec((H, L, D), lambda b, h: (0, 0, 0)),
        pl.BlockSpec((H, L, 1), lambda b, h: (0, 0, 0)),
    ]
    out_specs = (
        pl.BlockSpec((1, 1, 1, 1, TS), lambda b, h: (b // T, h, b % T, 0, 0)),
        pl.BlockSpec((1, 1, 1, 1, TS), lambda b, h: (b // T, h, b % T, 0, 0)),
        pl.BlockSpec((H, L, D), lambda b, h: (0, 0, 0)),
        pl.BlockSpec((1, 1, 1, 1, TS), lambda b, h: (b // T, h, b % T, 0, 0)),
        pl.BlockSpec((H, L, 1), lambda b, h: (0, 0, 0)),
    )
    return pl.pallas_call(
        _a_body,
        grid=grid,
        in_specs=in_specs,
        out_specs=out_specs,
        out_shape=out_shapes,
        scratch_shapes=[pltpu.VMEM((H, L, 1), jnp.float32),
                        pltpu.VMEM((TS, TS), jnp.float32),
                        pltpu.VMEM((1, TS), jnp.float32)],
    )(vecs, mask4, w, c_count.reshape(H, L, 1))


NROW = B * H * S
RPW = NROW // 32
CHUNK = 128
NCH = RPW // CHUNK
HL = H * L


def _make_sc_kernel():
    from jax.experimental.pallas import tpu_sc as plsc

    mesh = plsc.VectorSubcoreMesh(core_axis_name="c", subcore_axis_name="s")

    @functools.partial(
        pl.kernel,
        mesh=mesh,
        out_type=[
            jax.ShapeDtypeStruct((NROW, D), jnp.float32),
            jax.ShapeDtypeStruct((2, HL, D), jnp.float32),
        ],
        scratch_types=[
            pltpu.VMEM((NCH, CHUNK), jnp.int32),
            pltpu.VMEM((4, CHUNK, D), jnp.float32),
            pltpu.VMEM((4, CHUNK, D), jnp.float32),
            pltpu.VMEM((CHUNK, D), jnp.float32),
            pltpu.VMEM_SHARED((HL, D), jnp.float32),
            pltpu.SemaphoreType.DMA,
            pltpu.SemaphoreType.DMA,
            pltpu.SemaphoreType.DMA,
            pltpu.SemaphoreType.DMA,
        ],
        compiler_params=pltpu.CompilerParams(use_tc_tiling_on_sc=False),
    )
    def sc_b(gidx_hbm, cb_hbm, vecs_hbm, vh_hbm, parts_hbm,
             idx_v, v_buf, cz_buf, zbuf, acc, sem_v, sem_g, sem_s, sem_a):
        c = lax.axis_index("c")
        s = lax.axis_index("s")
        wid = s * 2 + c
        base = wid * RPW

        def _zero_row(i, _):
            r = i // (D // 16)
            col = (i % (D // 16)) * 16
            zbuf[r, pl.ds(col, 16)] = jnp.zeros((16,), jnp.float32)
            return _
        lax.fori_loop(0, CHUNK * (D // 16), _zero_row, 0)
        pltpu.sync_copy(gidx_hbm.at[pl.ds(wid * NCH, NCH)], idx_v)
        for k in range(HL // (16 * CHUNK)):
            pltpu.sync_copy(zbuf, acc.at[pl.ds(s * (HL // 16) + k * CHUNK, CHUNK)])
        plsc.subcore_barrier()

        prev = []
        for batch in range(NCH // 4):
            for cpy in prev:
                cpy.wait()
            prev = []
            vc, gc = [], []
            for j4 in range(4):
                j = batch * 4 + j4
                row0 = base + j * CHUNK
                vc.append(pltpu.async_copy(vecs_hbm.at[pl.ds(row0, CHUNK)],
                                           v_buf.at[j4], sem_v))
                gc.append(pltpu.async_copy(cb_hbm.at[idx_v.at[j]],
                                           cz_buf.at[j4], sem_g))
            for j4 in range(4):
                j = batch * 4 + j4
                row0 = base + j * CHUNK
                gc[j4].wait()
                prev.append(pltpu.async_copy(cz_buf.at[j4],
                                             vh_hbm.at[pl.ds(row0, CHUNK)],
                                             sem_s))
                vc[j4].wait()
                prev.append(pltpu.async_copy(v_buf.at[j4], acc.at[idx_v.at[j]],
                                             sem_a, add=True))
        for cpy in prev:
            cpy.wait()
        plsc.subcore_barrier()

        pltpu.sync_copy(acc.at[pl.ds(s * (HL // 16), HL // 16)],
                        parts_hbm.at[c, pl.ds(s * (HL // 16), HL // 16)])

    return sc_b


def _c_body(errs_ref, maskbh_ref, w2_ref, cc_ref, parts_ref, chat_ref,
            lcommit_ref, lcb_ref):
    lc = jnp.sum(maskbh_ref[...] * errs_ref[...]) * (1.0 / (B * S))
    lcommit_ref[...] = lc.reshape(1, 1)
    csum = parts_ref[0] + parts_ref[1]
    w2 = w2_ref[...]
    tgt = (1.0 - C_GAMMA) * w2 + C_GAMMA * csum
    s1 = jnp.sum(jnp.square(w2 - tgt))
    cc = cc_ref[...]
    ctgt = (1.0 - C_GAMMA) * cc + C_GAMMA * chat_ref[...]
    s2 = jnp.sum(jnp.square(cc - ctgt))
    lcb_ref[...] = (s1 + s2).reshape(1, 1)


def _run_c(errs2d, maskbh, w2, c_count, parts, c_count_hat):
    return pl.pallas_call(
        _c_body,
        out_shape=(jax.ShapeDtypeStruct((1, 1), jnp.float32),
                   jax.ShapeDtypeStruct((1, 1), jnp.float32)),
    )(errs2d, maskbh, w2, c_count, parts, c_count_hat)


def kernel(vecs, loss_mask, w, c_count):
    mask4 = loss_mask.reshape(B, T, 1, TS)
    z5, errs5, cb, gidx5, counts3 = _run_a(vecs, mask4, w, c_count)
    z = z5.reshape(B, H, S)
    c_count_hat = counts3.reshape(H, L)

    gidx2d = gidx5.reshape(NROW // CHUNK, CHUNK)
    vh_flat, parts = _make_sc_kernel()(gidx2d, cb.reshape(HL, D),
                                       vecs.reshape(NROW, D))
    vecs_hat = vh_flat.reshape(B, H, S, D)

    errs2d = errs5.reshape(B * H, S)
    maskbh = jnp.broadcast_to(loss_mask[:, None, :], (B, H, S)).reshape(B * H, S)
    lc, lcb = _run_c(errs2d, maskbh, w.reshape(HL, D), c_count,
                     parts, c_count_hat)
    return (vecs_hat, z, lc.reshape(()), lcb.reshape(()))
